# baseline (speedup 1.0000x reference)
import jax
import jax.numpy as jnp
from jax.experimental import pallas as pl
from jax.experimental.pallas import tpu as pltpu

_NSTEP = 4


def _gcn3_kernel(adj_ref, x_ref, w1_ref, b1_ref, w2_ref, b2_ref, w3_ref,
                 b3_ref, out_ref, s_ref, deg_ref, h1_ref):
    k = pl.program_id(0)
    blk_rows = adj_ref.shape[0]
    blk = (adj_ref[...] != 0).astype(jnp.bfloat16)
    s_ref[pl.ds(k * blk_rows, blk_rows), :] = blk
    ones = jnp.ones((blk_rows, 1), jnp.bfloat16)
    part = jax.lax.dot_general(
        blk, ones, (((0,), (0,)), ((), ())), preferred_element_type=jnp.float32)

    @pl.when(k == 0)
    def _():
        deg_ref[...] = 1.0 + part
        h1_ref[...] = jnp.dot(x_ref[...], w1_ref[...],
                              preferred_element_type=jnp.float32)

    @pl.when(jnp.logical_and(k > 0, k < _NSTEP - 1))
    def _():
        deg_ref[...] += part

    @pl.when(k == _NSTEP - 1)
    def _():
        dis = jax.lax.rsqrt(deg_ref[...] + part)
        dis2 = dis * dis
        S = s_ref[...]
        h = h1_ref[...]
        x = None
        for w_ref, b_ref in ((None, b1_ref), (w2_ref, b2_ref),
                             (w3_ref, b3_ref)):
            if x is not None:
                h = jnp.dot(x, w_ref[...], preferred_element_type=jnp.float32)
            y = dis * h
            agg = jax.lax.dot_general(
                S, y, (((0,), (0,)), ((), ())),
                preferred_element_type=jnp.float32)
            x = jnp.maximum(dis * agg + dis2 * h + b_ref[...], 0.0)
        out_ref[...] = x


def kernel(bone_features, bone_adj, W1, b1, W2, b2, W3, b3):
    n, d = bone_features.shape
    d_out = W3.shape[1]
    blk_rows = n // _NSTEP
    full = lambda shape: pl.BlockSpec(shape, lambda k: (0, 0))
    return pl.pallas_call(
        _gcn3_kernel,
        grid=(_NSTEP,),
        in_specs=[
            pl.BlockSpec((blk_rows, n), lambda k: (k, 0)),
            full((n, d)),
            full(W1.shape), full((1, W1.shape[1])),
            full(W2.shape), full((1, W2.shape[1])),
            full(W3.shape), full((1, W3.shape[1])),
        ],
        out_specs=full((n, d_out)),
        out_shape=jax.ShapeDtypeStruct((n, d_out), jnp.float32),
        scratch_shapes=[
            pltpu.VMEM((n, n), jnp.bfloat16),
            pltpu.VMEM((n, 1), jnp.float32),
            pltpu.VMEM((n, W1.shape[1]), jnp.float32),
        ],
    )(bone_adj, bone_features,
      W1, b1.reshape(1, -1), W2, b2.reshape(1, -1), W3, b3.reshape(1, -1))


# speedup vs baseline: 1.1347x; 1.1347x over previous
import jax
import jax.numpy as jnp
from jax.experimental import pallas as pl
from jax.experimental.pallas import tpu as pltpu

_NSTEP = 2


def _gcn3_kernel(adj_ref, x_ref, w1_ref, b1_ref, w2_ref, b2_ref, w3_ref,
                 b3_ref, out_ref, s_ref, deg_ref, h1_ref):
    k = pl.program_id(0)
    blk_rows = adj_ref.shape[0]
    blk = (adj_ref[...] != 0).astype(jnp.bfloat16)
    s_ref[pl.ds(k * blk_rows, blk_rows), :] = blk
    ones = jnp.ones((blk_rows, 1), jnp.bfloat16)
    part = jax.lax.dot_general(
        blk, ones, (((0,), (0,)), ((), ())), preferred_element_type=jnp.float32)

    @pl.when(k == 0)
    def _():
        deg_ref[...] = 1.0 + part
        h1_ref[...] = jnp.dot(x_ref[...], w1_ref[...],
                              preferred_element_type=jnp.float32)

    @pl.when(k == _NSTEP - 1)
    def _():
        dis = jax.lax.rsqrt(deg_ref[...] + part)
        dis2 = dis * dis
        S = s_ref[...]
        h = h1_ref[...]
        x = None
        for w_ref, b_ref in ((None, b1_ref), (w2_ref, b2_ref),
                             (w3_ref, b3_ref)):
            if x is not None:
                h = jnp.dot(x, w_ref[...], preferred_element_type=jnp.float32)
            y = dis * h
            agg = jax.lax.dot_general(
                S, y, (((0,), (0,)), ((), ())),
                preferred_element_type=jnp.float32)
            x = jnp.maximum(dis * agg + dis2 * h + b_ref[...], 0.0)
        out_ref[...] = x


def kernel(bone_features, bone_adj, W1, b1, W2, b2, W3, b3):
    n, d = bone_features.shape
    d_out = W3.shape[1]
    blk_rows = n // _NSTEP
    full = lambda shape: pl.BlockSpec(shape, lambda k: (0, 0))
    return pl.pallas_call(
        _gcn3_kernel,
        grid=(_NSTEP,),
        in_specs=[
            pl.BlockSpec((blk_rows, n), lambda k: (k, 0)),
            full((n, d)),
            full(W1.shape), full((1, W1.shape[1])),
            full(W2.shape), full((1, W2.shape[1])),
            full(W3.shape), full((1, W3.shape[1])),
        ],
        out_specs=full((n, d_out)),
        out_shape=jax.ShapeDtypeStruct((n, d_out), jnp.float32),
        scratch_shapes=[
            pltpu.VMEM((n, n), jnp.bfloat16),
            pltpu.VMEM((n, 1), jnp.float32),
            pltpu.VMEM((n, W1.shape[1]), jnp.float32),
        ],
    )(bone_adj, bone_features,
      W1, b1.reshape(1, -1), W2, b2.reshape(1, -1), W3, b3.reshape(1, -1))


# speedup vs baseline: 1.1689x; 1.0301x over previous
import jax
import jax.numpy as jnp
from jax.experimental import pallas as pl
from jax.experimental.pallas import tpu as pltpu

_NSTEP = 2


def _gcn3_kernel(adj_ref, x_ref, w1_ref, b1_ref, w2_ref, b2_ref, w3_ref,
                 b3_ref, out_ref, s_ref, deg_ref, h1_ref):
    k = pl.program_id(0)
    blk_rows = adj_ref.shape[0]
    blk = (adj_ref[...] != 0).astype(jnp.bfloat16)
    s_ref[:, pl.ds(k * blk_rows, blk_rows)] = blk.T
    ones = jnp.ones((blk_rows, 1), jnp.bfloat16)
    part = jax.lax.dot_general(
        blk, ones, (((0,), (0,)), ((), ())), preferred_element_type=jnp.float32)

    @pl.when(k == 0)
    def _():
        deg_ref[...] = 1.0 + part
        h1_ref[...] = jnp.dot(x_ref[...], w1_ref[...],
                              preferred_element_type=jnp.float32)

    @pl.when(k == _NSTEP - 1)
    def _():
        dis = jax.lax.rsqrt(deg_ref[...] + part)
        dis2 = dis * dis
        S = s_ref[...]
        h = h1_ref[...]
        x = None
        for w_ref, b_ref in ((None, b1_ref), (w2_ref, b2_ref),
                             (w3_ref, b3_ref)):
            if x is not None:
                h = jnp.dot(x, w_ref[...], preferred_element_type=jnp.float32)
            y = dis * h
            agg = jnp.dot(S, y, preferred_element_type=jnp.float32)
            x = jnp.maximum(dis * agg + dis2 * h + b_ref[...], 0.0)
        out_ref[...] = x


def kernel(bone_features, bone_adj, W1, b1, W2, b2, W3, b3):
    n, d = bone_features.shape
    d_out = W3.shape[1]
    blk_rows = n // _NSTEP
    full = lambda shape: pl.BlockSpec(shape, lambda k: (0, 0))
    return pl.pallas_call(
        _gcn3_kernel,
        grid=(_NSTEP,),
        in_specs=[
            pl.BlockSpec((blk_rows, n), lambda k: (k, 0)),
            full((n, d)),
            full(W1.shape), full((1, W1.shape[1])),
            full(W2.shape), full((1, W2.shape[1])),
            full(W3.shape), full((1, W3.shape[1])),
        ],
        out_specs=full((n, d_out)),
        out_shape=jax.ShapeDtypeStruct((n, d_out), jnp.float32),
        scratch_shapes=[
            pltpu.VMEM((n, n), jnp.bfloat16),
            pltpu.VMEM((n, 1), jnp.float32),
            pltpu.VMEM((n, W1.shape[1]), jnp.float32),
        ],
    )(bone_adj, bone_features,
      W1, b1.reshape(1, -1), W2, b2.reshape(1, -1), W3, b3.reshape(1, -1))
